# Initial kernel scaffold; baseline (speedup 1.0000x reference)
import jax
import jax.numpy as jnp
from jax.experimental import pallas as pl


def kernel(_input):
    raise NotImplementedError("write your pallas kernel here")


# speedup vs baseline: 16.4197x; 16.4197x over previous
import jax
import jax.numpy as jnp
import numpy as np
from jax import lax
from jax.experimental import pallas as pl
from jax.experimental.pallas import tpu as pltpu, tpu_sc as plsc

_INPUT = 224
_NSEL = 7
_GRID0 = 16
_GSTEP = 32
_B, _C = 32, 16

_NW = 32
_IPW = (_B * _C) // _NW
_RPW = _IPW * _NSEL
_OPW = _RPW * _NSEL
_L = 16

_ROW_OFF = [(t // _NSEL) * _INPUT + _GRID0 + _GSTEP * (t % _NSEL)
            for t in range(_RPW)]
_RI_TAB = np.array([e // _NSEL for e in range(_OPW)], dtype=np.int32)
_CI_TAB = np.array([_GRID0 + _GSTEP * (e % _NSEL) for e in range(_OPW)],
                   dtype=np.int32)

_mesh = plsc.VectorSubcoreMesh(core_axis_name="c", subcore_axis_name="s")


def _select_center(x2d, ri_tab, ci_tab):
    @pl.kernel(
        out_type=jax.ShapeDtypeStruct((_B * _C * _NSEL * _NSEL,), jnp.float32),
        mesh=_mesh,
        compiler_params=pltpu.CompilerParams(needs_layout_passes=False),
        scratch_types=[
            pltpu.VMEM((_RPW, _INPUT), jnp.float32),
            pltpu.VMEM((_OPW,), jnp.float32),
            pltpu.VMEM((_OPW,), jnp.int32),
            pltpu.VMEM((_OPW,), jnp.int32),
            pltpu.SemaphoreType.DMA,
        ],
    )
    def body(x_hbm, ri_hbm, ci_hbm, out_hbm, rows_v, out_v, ri_v, ci_v, sem):
        wid = lax.axis_index("s") * 2 + lax.axis_index("c")
        pltpu.sync_copy(ri_hbm, ri_v)
        pltpu.sync_copy(ci_hbm, ci_v)
        base = wid * (_IPW * _INPUT)
        descs = [
            pltpu.async_copy(
                x_hbm.at[pl.ds(base + _ROW_OFF[t], 1)],
                rows_v.at[pl.ds(t, 1)],
                sem,
            )
            for t in range(_RPW)
        ]
        for d in descs:
            d.wait()
        for k in range(_OPW // _L):
            riv = ri_v[pl.ds(_L * k, _L)]
            civ = ci_v[pl.ds(_L * k, _L)]
            out_v[pl.ds(_L * k, _L)] = plsc.load_gather(rows_v, [riv, civ])
        pltpu.sync_copy(out_v, out_hbm.at[pl.ds(wid * _OPW, _OPW)])

    return body(x2d, ri_tab, ci_tab)


def kernel(_input):
    x2d = _input.reshape(_B * _C * _INPUT, _INPUT)
    flat = _select_center(x2d, jnp.asarray(_RI_TAB), jnp.asarray(_CI_TAB))
    return flat.reshape(_B, _C, _NSEL, _NSEL)


# speedup vs baseline: 19.5884x; 1.1930x over previous
import jax
import jax.numpy as jnp
from jax import lax
from jax.experimental import pallas as pl
from jax.experimental.pallas import tpu as pltpu, tpu_sc as plsc

_INPUT = 224
_NSEL = 7
_GRID0 = 16
_GSTEP = 32
_B, _C = 32, 16
_NIMG = _B * _C

_NW = 32
_IPW = _NIMG // _NW
_OPW = _IPW * _NSEL * _NSEL
_L = 16
_MAGIC49, _MAGIC7, _SHIFT = 5350, 37450, 18

_mesh = plsc.VectorSubcoreMesh(core_axis_name="c", subcore_axis_name="s")


def _select_center(x4d):
    @pl.kernel(
        out_type=jax.ShapeDtypeStruct((_NIMG * _NSEL * _NSEL,), jnp.float32),
        mesh=_mesh,
        compiler_params=pltpu.CompilerParams(needs_layout_passes=False),
        scratch_types=[
            pltpu.VMEM((_IPW, _NSEL, 1, _INPUT), jnp.float32),
            pltpu.VMEM((_OPW,), jnp.float32),
            pltpu.SemaphoreType.DMA,
        ],
    )
    def body(x_hbm, out_hbm, rows_v, out_v, sem):
        wid = lax.axis_index("s") * 2 + lax.axis_index("c")
        img0 = wid * _IPW
        cp = pltpu.async_copy(
            x_hbm.at[pl.ds(img0, _IPW), :, pl.ds(_GRID0, 1), :],
            rows_v,
            sem,
        )
        lane = lax.iota(jnp.int32, _L)
        zero = lane - lane
        cp.wait()
        for k in range(_OPW // _L):
            e = lane + (_L * k)
            img = lax.shift_right_logical(e * _MAGIC49, _SHIFT)
            rem = e - 49 * img
            i = lax.shift_right_logical(rem * _MAGIC7, _SHIFT)
            ci = _GRID0 + _GSTEP * (rem - _NSEL * i)
            out_v[pl.ds(_L * k, _L)] = plsc.load_gather(
                rows_v, [img, i, zero, ci])
        pltpu.sync_copy(out_v, out_hbm.at[pl.ds(wid * _OPW, _OPW)])

    return body(x4d)


def kernel(_input):
    x4d = _input.reshape(_NIMG, _NSEL, _GSTEP, _INPUT)
    flat = _select_center(x4d)
    return flat.reshape(_B, _C, _NSEL, _NSEL)
